# baseline (speedup 1.0000x reference)
import dataclasses

import jax
import jax.numpy as jnp
from jax import lax
from jax.experimental import pallas as pl
from jax.experimental.pallas import tpu as pltpu
from jax.experimental.pallas import tpu_sc as plsc

_K = 20
_B = 128
_N = 8192
_NC = 2
_NS = 16
_NW = _NC * _NS
_RPW = _B // _NW
_CSZ = 64
_NCHUNK = _N // _CSZ
_VPC = _CSZ // 16
_NSUP = _NCHUNK // 16
_PAD = 1 << 20
_BIG = 1 << 30


def _store1(ref, pos, val, iota16):
    plsc.store_scatter(
        ref,
        [jnp.broadcast_to(pos, (16,))],
        jnp.broadcast_to(val, (16,)),
        mask=iota16 == 0,
    )


def _sort16(v, descending=False):
    s, _ = plsc.sort_key_val(v, v, descending=descending)
    return s


def _sc_topk_body(z_hbm, idx_hbm, b0, b1, b2, b3, cm_v, cm2_v, out_v,
                  s0, s1, s2, s3):
    wid = lax.axis_index("s") * _NC + lax.axis_index("c")
    iota16 = lax.broadcasted_iota(jnp.int32, (16,), 0)
    bufs = (b0, b1, b2, b3)
    sems = (s0, s1, s2, s3)
    row0 = wid * _RPW

    copies = [
        pltpu.async_copy(z_hbm.at[row0 + r], bufs[r], sems[r])
        for r in range(_RPW)
    ]
    for q in range(8):
        out_v[pl.ds(q * 16, 16)] = jnp.full((16,), _PAD, jnp.int32)

    for r in range(_RPW):
        buf = bufs[r]
        copies[r].wait()

        @pl.loop(0, _NCHUNK)
        def _(c):
            base = c * _CSZ
            m = jnp.abs(buf[pl.ds(base, 16)])
            for j in range(1, _VPC):
                m = jnp.maximum(m, jnp.abs(buf[pl.ds(base + j * 16, 16)]))
            _store1(cm_v, c, jnp.max(m), iota16)

        cm2_v[pl.ds(0, 16)] = jnp.full((16,), -1.0, jnp.float32)
        for s in range(_NSUP):
            _store1(cm2_v, s, jnp.max(cm_v[pl.ds(s * 16, 16)]), iota16)

        @pl.loop(0, _K)
        def _(k):
            c2 = cm2_v[pl.ds(0, 16)]
            gmax = jnp.max(c2)
            sstar = jnp.min(jnp.where(c2 == gmax, iota16, _BIG))
            cmg = cm_v[pl.ds(sstar * 16, 16)]
            lane = jnp.min(jnp.where(cmg == gmax, iota16, _BIG))
            cstar = sstar * 16 + lane
            base = cstar * _CSZ
            best = jnp.full((16,), _BIG, jnp.int32)
            for j in range(_VPC):
                v = jnp.abs(buf[pl.ds(base + j * 16, 16)])
                best = jnp.minimum(
                    best,
                    jnp.where(v == gmax, base + j * 16 + iota16, _BIG),
                )
            pos = jnp.min(best)
            _store1(out_v, r * 32 + k, pos, iota16)
            _store1(buf, pos, jnp.float32(jnp.nan), iota16)
            m = jnp.full((16,), -1.0, jnp.float32)
            for j in range(_VPC):
                v = jnp.abs(buf[pl.ds(base + j * 16, 16)])
                m = jnp.where(v > m, v, m)
            _store1(cm_v, cstar, jnp.max(m), iota16)
            _store1(
                cm2_v, sstar, jnp.max(cm_v[pl.ds(sstar * 16, 16)]), iota16
            )

        a = _sort16(out_v[pl.ds(r * 32, 16)])
        b = _sort16(out_v[pl.ds(r * 32 + 16, 16)], descending=True)
        lo = jnp.minimum(a, b)
        hi = jnp.maximum(a, b)
        out_v[pl.ds(r * 32, 16)] = _sort16(lo)
        out_v[pl.ds(r * 32 + 16, 16)] = _sort16(hi)

    pltpu.sync_copy(out_v, idx_hbm.at[pl.ds(row0 * 32, _RPW * 32)])


def _sc_topk(z):
    mesh = plsc.VectorSubcoreMesh(core_axis_name="c", subcore_axis_name="s")
    cp = pltpu.CompilerParams()
    if "needs_layout_passes" in pltpu.CompilerParams.__dataclass_fields__:
        cp = dataclasses.replace(cp, needs_layout_passes=False)
    return pl.kernel(
        _sc_topk_body,
        out_type=jax.ShapeDtypeStruct((_B * 32,), jnp.int32),
        mesh=mesh,
        compiler_params=cp,
        scratch_types=[
            pltpu.VMEM((_N,), jnp.float32),
            pltpu.VMEM((_N,), jnp.float32),
            pltpu.VMEM((_N,), jnp.float32),
            pltpu.VMEM((_N,), jnp.float32),
            pltpu.VMEM((_NCHUNK,), jnp.float32),
            pltpu.VMEM((16,), jnp.float32),
            pltpu.VMEM((_RPW * 32,), jnp.int32),
            pltpu.SemaphoreType.DMA,
            pltpu.SemaphoreType.DMA,
            pltpu.SemaphoreType.DMA,
            pltpu.SemaphoreType.DMA,
        ],
    )(z)


def _labels_body(idx_ref, labels_ref, ones_ref):
    idx = idx_ref[...]
    idx_t = idx.T
    acc = idx[:, 0:1] == idx_t[0:1, :]
    for k in range(1, _K):
        acc = acc & (idx[:, k : k + 1] == idx_t[k : k + 1, :])
    labels_ref[...] = acc.astype(jnp.float32)
    ones_ref[...] = jnp.ones((_B, _B), jnp.float32)


def kernel(z):
    idx = _sc_topk(z).reshape(_B, 32)
    labels, ones = pl.pallas_call(
        _labels_body,
        out_shape=(
            jax.ShapeDtypeStruct((_B, _B), jnp.float32),
            jax.ShapeDtypeStruct((_B, _B), jnp.float32),
        ),
    )(idx)
    return labels, ones


# speedup vs baseline: 1.0156x; 1.0156x over previous
import dataclasses

import jax
import jax.numpy as jnp
from jax import lax
from jax.experimental import pallas as pl
from jax.experimental.pallas import tpu as pltpu
from jax.experimental.pallas import tpu_sc as plsc

_K = 20
_B = 128
_N = 8192
_NC = 2
_NS = 16
_NW = _NC * _NS
_RPW = _B // _NW
_CSZ = 64
_NCHUNK = _N // _CSZ
_VPC = _CSZ // 16
_NSUP = _NCHUNK // 16
_PAD = 1 << 20
_BIG = 1 << 30


def _store1(ref, pos, val, iota16):
    plsc.store_scatter(
        ref,
        [jnp.broadcast_to(pos, (16,))],
        jnp.broadcast_to(val, (16,)),
        mask=iota16 == 0,
    )


def _sort16(v, descending=False):
    s, _ = plsc.sort_key_val(v, v, descending=descending)
    return s


def _sc_topk_body(z_hbm, idx_hbm,
                  b0, b1, b2, b3,
                  cm0, cm1, cm2, cm3,
                  g0, g1, g2, g3,
                  o0, o1, o2, o3,
                  s0, s1, s2, s3):
    wid = lax.axis_index("s") * _NC + lax.axis_index("c")
    iota16 = lax.broadcasted_iota(jnp.int32, (16,), 0)
    bufs = (b0, b1, b2, b3)
    cms = (cm0, cm1, cm2, cm3)
    sups = (g0, g1, g2, g3)
    outs = (o0, o1, o2, o3)
    sems = (s0, s1, s2, s3)
    row0 = wid * _RPW

    copies = [
        pltpu.async_copy(z_hbm.at[row0 + r], bufs[r], sems[r])
        for r in range(_RPW)
    ]
    for r in range(_RPW):
        outs[r][pl.ds(0, 16)] = jnp.full((16,), _PAD, jnp.int32)
        outs[r][pl.ds(16, 16)] = jnp.full((16,), _PAD, jnp.int32)

    for r in range(_RPW):
        buf = bufs[r]
        cm_v = cms[r]
        sup_v = sups[r]
        copies[r].wait()

        @pl.loop(0, _NCHUNK)
        def _(c):
            base = c * _CSZ
            m = jnp.abs(buf[pl.ds(base, 16)])
            for j in range(1, _VPC):
                m = jnp.maximum(m, jnp.abs(buf[pl.ds(base + j * 16, 16)]))
            _store1(cm_v, c, jnp.max(m), iota16)

        sup_v[pl.ds(0, 16)] = jnp.full((16,), -1.0, jnp.float32)
        for s in range(_NSUP):
            _store1(sup_v, s, jnp.max(cm_v[pl.ds(s * 16, 16)]), iota16)

    @pl.loop(0, _K)
    def _(k):
        for r in range(_RPW):
            buf = bufs[r]
            cm_v = cms[r]
            sup_v = sups[r]
            c2 = sup_v[pl.ds(0, 16)]
            gmax = jnp.max(c2)
            sstar = jnp.min(jnp.where(c2 == gmax, iota16, _BIG))
            cmg = cm_v[pl.ds(sstar * 16, 16)]
            lane = jnp.min(jnp.where(cmg == gmax, iota16, _BIG))
            cstar = sstar * 16 + lane
            base = cstar * _CSZ
            best = jnp.full((16,), _BIG, jnp.int32)
            for j in range(_VPC):
                v = jnp.abs(buf[pl.ds(base + j * 16, 16)])
                best = jnp.minimum(
                    best,
                    jnp.where(v == gmax, base + j * 16 + iota16, _BIG),
                )
            pos = jnp.min(best)
            _store1(outs[r], k, pos, iota16)
            _store1(buf, pos, jnp.float32(jnp.nan), iota16)
            m = jnp.full((16,), -1.0, jnp.float32)
            for j in range(_VPC):
                v = jnp.abs(buf[pl.ds(base + j * 16, 16)])
                m = jnp.where(v > m, v, m)
            _store1(cm_v, cstar, jnp.max(m), iota16)
            _store1(
                sup_v, sstar, jnp.max(cm_v[pl.ds(sstar * 16, 16)]), iota16
            )

    for r in range(_RPW):
        a = _sort16(outs[r][pl.ds(0, 16)])
        b = _sort16(outs[r][pl.ds(16, 16)], descending=True)
        lo = jnp.minimum(a, b)
        hi = jnp.maximum(a, b)
        outs[r][pl.ds(0, 16)] = _sort16(lo)
        outs[r][pl.ds(16, 16)] = _sort16(hi)
    for r in range(_RPW):
        pltpu.sync_copy(outs[r], idx_hbm.at[pl.ds((row0 + r) * 32, 32)])


def _sc_topk(z):
    mesh = plsc.VectorSubcoreMesh(core_axis_name="c", subcore_axis_name="s")
    cp = pltpu.CompilerParams()
    if "needs_layout_passes" in pltpu.CompilerParams.__dataclass_fields__:
        cp = dataclasses.replace(cp, needs_layout_passes=False)
    return pl.kernel(
        _sc_topk_body,
        out_type=jax.ShapeDtypeStruct((_B * 32,), jnp.int32),
        mesh=mesh,
        compiler_params=cp,
        scratch_types=(
            [pltpu.VMEM((_N,), jnp.float32)] * _RPW
            + [pltpu.VMEM((_NCHUNK,), jnp.float32)] * _RPW
            + [pltpu.VMEM((16,), jnp.float32)] * _RPW
            + [pltpu.VMEM((32,), jnp.int32)] * _RPW
            + [pltpu.SemaphoreType.DMA] * _RPW
        ),
    )(z)


def _labels_body(idx_ref, labels_ref, ones_ref):
    idx = idx_ref[...]
    idx_t = idx.T
    acc = idx[:, 0:1] == idx_t[0:1, :]
    for k in range(1, _K):
        acc = acc & (idx[:, k : k + 1] == idx_t[k : k + 1, :])
    labels_ref[...] = acc.astype(jnp.float32)
    ones_ref[...] = jnp.ones((_B, _B), jnp.float32)


def kernel(z):
    idx = _sc_topk(z).reshape(_B, 32)
    labels, ones = pl.pallas_call(
        _labels_body,
        out_shape=(
            jax.ShapeDtypeStruct((_B, _B), jnp.float32),
            jax.ShapeDtypeStruct((_B, _B), jnp.float32),
        ),
    )(idx)
    return labels, ones


# speedup vs baseline: 1.1992x; 1.1807x over previous
import dataclasses

import jax
import jax.numpy as jnp
from jax import lax
from jax.experimental import pallas as pl
from jax.experimental.pallas import tpu as pltpu
from jax.experimental.pallas import tpu_sc as plsc

_K = 20
_B = 128
_N = 8192
_NC = 2
_NS = 16
_NW = _NC * _NS
_RPW = _B // _NW
_NCHUNK = 64
_CSZ = _N // _NCHUNK
_VPC = _CSZ // 16
_PAD = 1 << 20
_BIG = 1 << 30


def _store1(ref, pos, val, iota16):
    plsc.store_scatter(
        ref,
        [jnp.broadcast_to(pos, (16,))],
        jnp.broadcast_to(val, (16,)),
        mask=iota16 == 0,
    )


def _sort16(v, descending=False):
    s, _ = plsc.sort_key_val(v, v, descending=descending)
    return s


def _sc_topk_body(z_hbm, idx_hbm, b0, b1, b2, b3, cm_v, out_v,
                  s0, s1, s2, s3):
    wid = lax.axis_index("s") * _NC + lax.axis_index("c")
    iota16 = lax.broadcasted_iota(jnp.int32, (16,), 0)
    bufs = (b0, b1, b2, b3)
    sems = (s0, s1, s2, s3)
    row0 = wid * _RPW

    copies = [
        pltpu.async_copy(z_hbm.at[row0 + r], bufs[r], sems[r])
        for r in range(_RPW)
    ]

    out_v[pl.ds(0, 16)] = jnp.full((16,), _PAD, jnp.int32)
    out_v[pl.ds(16, 16)] = jnp.full((16,), _PAD, jnp.int32)

    for r in range(_RPW):
        buf = bufs[r]
        copies[r].wait()

        @pl.loop(0, _NCHUNK)
        def _(c):
            base = c * _CSZ
            m = jnp.full((16,), -1.0, jnp.float32)
            for j in range(_VPC):
                m = jnp.maximum(m, jnp.abs(buf[pl.ds(base + j * 16, 16)]))
            _store1(cm_v, c, jnp.max(m), iota16)

        @pl.loop(0, _K)
        def _(k):
            cms = [
                cm_v[pl.ds(16 * t, 16)] for t in range(_NCHUNK // 16)
            ]
            m01 = cms[0]
            for t in range(1, _NCHUNK // 16):
                m01 = jnp.maximum(m01, cms[t])
            gmax = jnp.max(m01)
            cand = jnp.full((16,), _BIG, jnp.int32)
            for t in range(_NCHUNK // 16):
                cand = jnp.minimum(
                    cand,
                    jnp.where(cms[t] == gmax, iota16 + 16 * t, _BIG),
                )
            cstar = jnp.min(cand)
            base = cstar * _CSZ
            best = jnp.full((16,), _BIG, jnp.int32)
            cnt = jnp.zeros((16,), jnp.int32)
            m_ex = jnp.full((16,), -1.0, jnp.float32)
            for j in range(_VPC):
                v = jnp.abs(buf[pl.ds(base + j * 16, 16)])
                ismax = v == gmax
                best = jnp.minimum(
                    best, jnp.where(ismax, base + j * 16 + iota16, _BIG)
                )
                cnt = cnt + ismax.astype(jnp.int32)
                nm = jnp.where(ismax, jnp.float32(-1.0), v)
                m_ex = jnp.where(nm > m_ex, nm, m_ex)
            pos = jnp.min(best)
            ntot = jnp.sum(cnt)
            newmax = jnp.where(ntot > 1, gmax, jnp.max(m_ex))
            _store1(out_v, k, pos, iota16)
            _store1(buf, pos, jnp.float32(jnp.nan), iota16)
            _store1(cm_v, cstar, newmax, iota16)

        a = _sort16(out_v[pl.ds(0, 16)])
        b = _sort16(out_v[pl.ds(16, 16)], descending=True)
        lo = jnp.minimum(a, b)
        hi = jnp.maximum(a, b)
        out_v[pl.ds(0, 16)] = _sort16(lo)
        out_v[pl.ds(16, 16)] = _sort16(hi)

        pltpu.sync_copy(out_v, idx_hbm.at[row0 + r])


def _sc_topk(z):
    mesh = plsc.VectorSubcoreMesh(core_axis_name="c", subcore_axis_name="s")
    cp = pltpu.CompilerParams()
    if "needs_layout_passes" in pltpu.CompilerParams.__dataclass_fields__:
        cp = dataclasses.replace(cp, needs_layout_passes=False)
    return pl.kernel(
        _sc_topk_body,
        out_type=jax.ShapeDtypeStruct((_B, 32), jnp.int32),
        mesh=mesh,
        compiler_params=cp,
        scratch_types=(
            [pltpu.VMEM((_N,), jnp.float32)] * _RPW
            + [
                pltpu.VMEM((_NCHUNK,), jnp.float32),
                pltpu.VMEM((32,), jnp.int32),
            ]
            + [pltpu.SemaphoreType.DMA] * _RPW
        ),
    )(z)


def _labels_body(idx_ref, labels_ref, ones_ref):
    idx = idx_ref[...]
    idx_t = idx.T
    acc = idx[:, 0:1] == idx_t[0:1, :]
    for k in range(1, _K):
        acc = acc & (idx[:, k : k + 1] == idx_t[k : k + 1, :])
    labels_ref[...] = acc.astype(jnp.float32)
    ones_ref[...] = jnp.ones((_B, _B), jnp.float32)


def kernel(z):
    idx = _sc_topk(z)
    labels, ones = pl.pallas_call(
        _labels_body,
        out_shape=(
            jax.ShapeDtypeStruct((_B, _B), jnp.float32),
            jax.ShapeDtypeStruct((_B, _B), jnp.float32),
        ),
    )(idx)
    return labels, ones
